# baseline (speedup 1.0000x reference)
import functools

import jax
import jax.numpy as jnp
from jax import lax
from jax.experimental import pallas as pl
from jax.experimental.pallas import tpu as pltpu
from jax.experimental.pallas import tpu_sc as plsc

_EMB = 64
_ARITY = 6
_RELW = 398

_NC, _NS = 2, 16
_NW = _NC * _NS


def _sc_gather(E_w, relT, eidx, ridx):
    n_e = eidx.shape[0]
    n_r = ridx.shape[0]
    epw = n_e // _NW
    rpw = n_r // _NW
    mesh = plsc.VectorSubcoreMesh(core_axis_name="c", subcore_axis_name="s")

    @functools.partial(
        pl.kernel,
        mesh=mesh,
        out_type=(
            jax.ShapeDtypeStruct((n_r, _ARITY * _EMB), jnp.float32),
            jax.ShapeDtypeStruct((n_r, _RELW), jnp.float32),
        ),
        scratch_types=[
            pltpu.VMEM((epw,), jnp.int32),
            pltpu.VMEM((rpw,), jnp.int32),
            pltpu.SemaphoreType.DMA,
            pltpu.SemaphoreType.DMA,
        ],
    )
    def k(E_hbm, relT_hbm, eidx_hbm, ridx_hbm, emb_out, rel_out,
          eidx_v, ridx_v, sem_e, sem_r):
        wid = lax.axis_index("s") * _NC + lax.axis_index("c")
        be = wid * epw
        br = wid * rpw
        pltpu.sync_copy(eidx_hbm.at[pl.ds(be, epw)], eidx_v)
        pltpu.sync_copy(ridx_hbm.at[pl.ds(br, rpw)], ridx_v)

        def e_body(g, carry):
            iv = eidx_v[pl.ds(g * 16, 16)]
            for t in range(16):
                j = be + g * 16 + t
                ib = j // _ARITY
                a = j - ib * _ARITY
                pltpu.async_copy(E_hbm.at[iv[t]],
                                 emb_out.at[ib, pl.ds(a * _EMB, _EMB)], sem_e)
            return carry

        def r_body(g, carry):
            rv = ridx_v[pl.ds(g * 16, 16)]
            for t in range(16):
                pltpu.async_copy(relT_hbm.at[rv[t]],
                                 rel_out.at[br + g * 16 + t], sem_r)
            return carry

        lax.fori_loop(0, epw // 16, e_body, 0)
        lax.fori_loop(0, rpw // 16, r_body, 0)
        pltpu.make_async_copy(emb_out.at[pl.ds(0, rpw)],
                              emb_out.at[pl.ds(0, rpw)], sem_e).wait()
        pltpu.make_async_copy(rel_out.at[pl.ds(0, rpw)],
                              rel_out.at[pl.ds(0, rpw)], sem_r).wait()

    return k(E_w, relT, eidx, ridx)


def _tc_body(emb_ref, rel_ref, out_ref):
    emb = emb_ref[...]
    rel = rel_ref[...]
    prod = emb * rel[:, : _ARITY * _EMB]
    pa = []
    for a in range(_ARITY):
        acc = prod[:, a * 64 : a * 64 + 8]
        for w in range(1, 8):
            acc = acc + prod[:, a * 64 + w * 8 : a * 64 + w * 8 + 8]
        pa.append(acc)
    P = [pa[0]]
    for a in range(1, _ARITY):
        P.append(P[-1] + pa[a])
    S = [pa[5]]
    for a in range(4, 0, -1):
        S.append(S[-1] + pa[a])
    S = S[::-1]
    rb = rel[:, 384:392]
    variants = [P[0], S[0], P[1], S[1], P[2], S[2], P[3], S[3], P[4], S[4], P[5]]
    s = [jnp.sum(jnp.tanh(v + rb), axis=1) for v in variants]
    out = (rel[:, 392] * s[0] * s[1]
           + rel[:, 393] * s[2] * s[3]
           + rel[:, 394] * s[4] * s[5]
           + rel[:, 395] * s[6] * s[7]
           + rel[:, 396] * s[8] * s[9]
           + rel[:, 397] * s[10])
    out_ref[...] = out


def _tc_compute(emb, rel):
    B = rel.shape[0]
    BT = 512
    return pl.pallas_call(
        _tc_body,
        grid=(B // BT,),
        in_specs=[
            pl.BlockSpec((BT, _ARITY * _EMB), lambda i: (i, 0)),
            pl.BlockSpec((BT, _RELW), lambda i: (i, 0)),
        ],
        out_specs=pl.BlockSpec((BT,), lambda i: (i,)),
        out_shape=jax.ShapeDtypeStruct((B,), jnp.float32),
    )(emb, rel)


def kernel(r_idx, e1_idx, e2_idx, e3_idx, e4_idx, e5_idx, e6_idx,
           E_w, R_w, R_bias_w, Rw0, Rw1, Rw2, Rw3, Rw4, Rw5):
    relT = jnp.concatenate(
        [R_w, R_bias_w, Rw0, Rw1, Rw2, Rw3, Rw4, Rw5], axis=1)
    eidx = jnp.stack(
        [e1_idx, e2_idx, e3_idx, e4_idx, e5_idx, e6_idx], axis=1
    ).reshape(-1).astype(jnp.int32)
    emb, rel = _sc_gather(E_w, relT, eidx, r_idx.astype(jnp.int32))
    return _tc_compute(emb, rel)


# speedup vs baseline: 1.3974x; 1.3974x over previous
import functools

import jax
import jax.numpy as jnp
from jax import lax
from jax.experimental import pallas as pl
from jax.experimental.pallas import tpu as pltpu
from jax.experimental.pallas import tpu_sc as plsc

_EMB = 64
_ARITY = 6
_RELW = 512

_NC, _NS = 2, 16
_NW = _NC * _NS
_BPW = 128


def _sc_gather(EP, relT, eidx, ridx):
    n_r = ridx.shape[0]
    mesh = plsc.VectorSubcoreMesh(core_axis_name="c", subcore_axis_name="s")

    @functools.partial(
        pl.kernel,
        mesh=mesh,
        out_type=(
            jax.ShapeDtypeStruct((n_r, _ARITY * 128), jnp.float32),
            jax.ShapeDtypeStruct((n_r, _RELW), jnp.float32),
        ),
        scratch_types=[
            pltpu.VMEM((_ARITY * _BPW,), jnp.int32),
            pltpu.VMEM((_BPW,), jnp.int32),
            pltpu.VMEM((2, _BPW, 128), jnp.float32),
            pltpu.VMEM((_BPW, _RELW), jnp.float32),
            pltpu.SemaphoreType.DMA,
            pltpu.SemaphoreType.DMA,
            pltpu.SemaphoreType.DMA,
        ],
    )
    def k(EP_hbm, relT_hbm, eidx_hbm, ridx_hbm, emb_out, rel_out,
          eidx_v, ridx_v, emb_v, rel_v, sem0, sem1, sem_r):
        wid = lax.axis_index("s") * _NC + lax.axis_index("c")
        be = wid * _ARITY * _BPW
        br = wid * _BPW
        pltpu.sync_copy(eidx_hbm.at[pl.ds(be, _ARITY * _BPW)], eidx_v)
        pltpu.sync_copy(ridx_hbm.at[pl.ds(br, _BPW)], ridx_v)
        cp_r = pltpu.async_copy(relT_hbm.at[ridx_v], rel_v, sem_r)
        sems = (sem0, sem1)

        cps = [pltpu.async_copy(EP_hbm.at[eidx_v.at[pl.ds(a * _BPW, _BPW)]],
                                emb_v.at[a % 2], sems[a % 2])
               for a in range(2)]
        for a in range(_ARITY):
            cps[a % 2].wait()
            pltpu.sync_copy(
                emb_v.at[a % 2],
                emb_out.at[pl.ds(br, _BPW), pl.ds(a * 128, 128)])
            if a + 2 < _ARITY:
                cps[a % 2] = pltpu.async_copy(
                    EP_hbm.at[eidx_v.at[pl.ds((a + 2) * _BPW, _BPW)]],
                    emb_v.at[a % 2], sems[a % 2])
        cp_r.wait()
        pltpu.sync_copy(rel_v, rel_out.at[pl.ds(br, _BPW)])

    return k(EP, relT, eidx, ridx)


def _tc_body(emb_ref, rel_ref, out_ref):
    emb = emb_ref[...]
    rel = rel_ref[...]
    pa = []
    for a in range(_ARITY):
        prod = emb[:, a * 128 : a * 128 + 64] * rel[:, a * 64 : a * 64 + 64]
        acc = prod[:, 0:8]
        for w in range(1, 8):
            acc = acc + prod[:, w * 8 : w * 8 + 8]
        pa.append(acc)
    P = [pa[0]]
    for a in range(1, _ARITY):
        P.append(P[-1] + pa[a])
    S = [pa[5]]
    for a in range(4, 0, -1):
        S.append(S[-1] + pa[a])
    S = S[::-1]
    rb = rel[:, 384:392]
    variants = [P[0], S[0], P[1], S[1], P[2], S[2], P[3], S[3], P[4], S[4], P[5]]
    s = [jnp.sum(jnp.tanh(v + rb), axis=1) for v in variants]
    out = (rel[:, 392] * s[0] * s[1]
           + rel[:, 393] * s[2] * s[3]
           + rel[:, 394] * s[4] * s[5]
           + rel[:, 395] * s[6] * s[7]
           + rel[:, 396] * s[8] * s[9]
           + rel[:, 397] * s[10])
    out_ref[...] = out


def _tc_compute(emb, rel):
    B = rel.shape[0]
    BT = 512
    return pl.pallas_call(
        _tc_body,
        grid=(B // BT,),
        in_specs=[
            pl.BlockSpec((BT, _ARITY * 128), lambda i: (i, 0)),
            pl.BlockSpec((BT, _RELW), lambda i: (i, 0)),
        ],
        out_specs=pl.BlockSpec((BT,), lambda i: (i,)),
        out_shape=jax.ShapeDtypeStruct((B,), jnp.float32),
    )(emb, rel)


def kernel(r_idx, e1_idx, e2_idx, e3_idx, e4_idx, e5_idx, e6_idx,
           E_w, R_w, R_bias_w, Rw0, Rw1, Rw2, Rw3, Rw4, Rw5):
    B = r_idx.shape[0]
    EP = jnp.pad(E_w, ((0, 0), (0, 128 - _EMB)))
    relT = jnp.concatenate(
        [R_w, R_bias_w, Rw0, Rw1, Rw2, Rw3, Rw4, Rw5,
         jnp.zeros((R_w.shape[0], _RELW - 398), jnp.float32)], axis=1)
    eidx = jnp.stack(
        [e1_idx, e2_idx, e3_idx, e4_idx, e5_idx, e6_idx], axis=0
    ).reshape(_ARITY, _NW, _BPW).transpose(1, 0, 2).reshape(-1).astype(jnp.int32)
    emb, rel = _sc_gather(EP, relT, eidx, r_idx.astype(jnp.int32))
    return _tc_compute(emb, rel)
